# Initial kernel scaffold; baseline (speedup 1.0000x reference)
import jax
import jax.numpy as jnp
from jax.experimental import pallas as pl


def kernel(x, knn_edge_index, ppi_edge_index, cols0_Wl, cols0_bl, cols0_Wr, cols1_Wl, cols1_bl, cols1_Wr, rows0_Wl, rows0_bl, rows0_Wr, rows1_Wl, rows1_bl, rows1_Wr, re_gcn_W, re_gcn_b, re_Wq, re_bq, re_Wk, re_bk, re_Wv, re_bv, re_Ws, re_bs, ce_gcn_W, ce_gcn_b, ce_mu_Wq, ce_mu_bq, ce_mu_Wk, ce_mu_bk, ce_mu_Wv, ce_mu_bv, ce_mu_Ws, ce_mu_bs, ce_ls_Wq, ce_ls_bq, ce_ls_Wk, ce_ls_bk, ce_ls_Wv, ce_ls_bv, ce_ls_Ws, ce_ls_bs, dec_W, dec_b, bn_gamma, bn_beta, transp_W, transp_b, comb_W, comb_b):
    raise NotImplementedError("write your pallas kernel here")


# speedup vs baseline: 4.6589x; 4.6589x over previous
import functools

import jax
import jax.numpy as jnp
from jax import lax
from jax.experimental import pallas as pl
from jax.experimental.pallas import tpu as pltpu
from jax.experimental.pallas import tpu_sc as plsc

R = 2000
C = 2048
NP = 2048
E_PPI = 80000
E_KNN = 30720
INTER = 512
EMBD = 128
SCALE_PARAM = 3.0
MAX_LOGSTD = 10.0

ROWS = 32
N_TILES = 32
N_PASS = NP // (ROWS * N_TILES)
CH_PPI = 4000
CH_KNN = 3840

_f32 = jnp.float32


def _adj_body(ksrc_ref, kdst_ref, psrc_ref, pdst_ref, zer_ref,
              aknn_ref, appi_ref, esrc, edst, acc):
    cid = lax.axis_index("c")
    sid = lax.axis_index("s")
    wid = sid * 2 + cid
    ones = jnp.ones((16,), _f32)

    def scan_graph(src_ref, dst_ref, out_ref, n_edges, ch):
        nch = n_edges // ch
        for p in range(N_PASS):
            base = (p * N_TILES + wid) * ROWS
            pltpu.sync_copy(zer_ref, acc)

            def chunk_body(ci, _):
                off = ci * ch
                pltpu.sync_copy(src_ref.at[pl.ds(off, ch)],
                                esrc.at[pl.ds(0, ch)])
                pltpu.sync_copy(dst_ref.at[pl.ds(off, ch)],
                                edst.at[pl.ds(0, ch)])

                def step(j, carry):
                    s16 = esrc[pl.ds(j * 16, 16)]
                    d16 = edst[pl.ds(j * 16, 16)]
                    locr = d16 - base
                    msk = (locr >= 0) & (locr < ROWS)
                    flat = jnp.where(msk, locr * NP + s16, 0)
                    plsc.addupdate_scatter(acc, [flat], ones, mask=msk)
                    return carry

                lax.fori_loop(0, ch // 16, step, 0)
                return _

            lax.fori_loop(0, nch, chunk_body, 0)
            pltpu.sync_copy(acc, out_ref.at[pl.ds(base * NP, ROWS * NP)])

    scan_graph(ksrc_ref, kdst_ref, aknn_ref, E_KNN, CH_KNN)
    scan_graph(psrc_ref, pdst_ref, appi_ref, E_PPI, CH_PPI)


def _build_adj(knn_ei, ppi_ei):
    mesh = plsc.VectorSubcoreMesh(core_axis_name="c", subcore_axis_name="s")
    zer = jnp.zeros((ROWS * NP,), _f32)
    call = pl.kernel(
        _adj_body,
        out_type=(jax.ShapeDtypeStruct((NP * NP,), _f32),
                  jax.ShapeDtypeStruct((NP * NP,), _f32)),
        mesh=mesh,
        compiler_params=pltpu.CompilerParams(needs_layout_passes=False),
        scratch_types=[
            pltpu.VMEM((CH_PPI,), jnp.int32),
            pltpu.VMEM((CH_PPI,), jnp.int32),
            pltpu.VMEM((ROWS * NP,), _f32),
        ],
    )
    a_knn, a_ppi = call(knn_ei[0], knn_ei[1], ppi_ei[0], ppi_ei[1], zer)
    return a_knn.reshape(NP, NP), a_ppi.reshape(NP, NP)


def _leaky(x):
    return jnp.where(x >= 0, x, 0.01 * x)


def _dot(a, b):
    return jnp.dot(a, b, preferred_element_type=_f32,
                   precision=lax.Precision.HIGHEST)


def _dot_nt(a, b):
    return lax.dot_general(a, b, (((1,), (1,)), ((), ())),
                           preferred_element_type=_f32,
                           precision=lax.Precision.HIGHEST)


BM = 256
GRID = NP // BM
_TC_PARAMS = pltpu.CompilerParams(vmem_limit_bytes=100 * 1024 * 1024)


def _sage_mean_tc(a_ref, xf_ref, o_ref):
    a = a_ref[...]
    cnt = jnp.sum(a, axis=1, keepdims=True)
    o_ref[...] = _dot(a, xf_ref[...]) / jnp.maximum(cnt, 1.0)


def _sage_comb_tc(m_ref, xb_ref, wl_ref, wr_ref, bl_ref, o_ref):
    acc = _dot(m_ref[...], wl_ref[...]) + bl_ref[...]
    acc = acc + _dot(xb_ref[...], wr_ref[...])
    o_ref[...] = _leaky(acc)


def _sage(A, X, Wl, bl, Wr):
    full = pl.BlockSpec((NP, NP), lambda i: (0, 0))
    blk = pl.BlockSpec((BM, NP), lambda i: (i, 0))
    mean = pl.pallas_call(
        _sage_mean_tc,
        grid=(GRID,),
        compiler_params=_TC_PARAMS,
        in_specs=[blk, full],
        out_specs=blk,
        out_shape=jax.ShapeDtypeStruct((NP, NP), _f32),
    )(A, X)
    return pl.pallas_call(
        _sage_comb_tc,
        grid=(GRID,),
        compiler_params=_TC_PARAMS,
        in_specs=[blk, blk, full, full,
                  pl.BlockSpec((1, NP), lambda i: (0, 0))],
        out_specs=blk,
        out_shape=jax.ShapeDtypeStruct((NP, NP), _f32),
    )(mean, X, Wl, Wr, bl)


def _gcn_tc(a_ref, h_ref, w_ref, b_ref, o_ref):
    a = a_ref[...]
    cnt = jnp.sum(a, axis=1, keepdims=True)
    dinv = lax.rsqrt(cnt + 1.0)
    g = _dot(h_ref[...], w_ref[...]) * dinv
    o_ref[...] = _leaky((_dot(a, g) + g) * dinv + b_ref[...])


def _gcn(A, H, W, b):
    return pl.pallas_call(
        _gcn_tc,
        compiler_params=_TC_PARAMS,
        out_shape=jax.ShapeDtypeStruct((NP, INTER), _f32),
    )(A, H, W, b)


def _lin4_tc(r_ref, wq, bq, wk, bk, wv, bv, ws, bs, oq, ok, ov, os):
    r = r_ref[...]
    oq[...] = _dot(r, wq[...]) + bq[...]
    ok[...] = _dot(r, wk[...]) + bk[...]
    ov[...] = _dot(r, wv[...]) + bv[...]
    os[...] = _dot(r, ws[...]) + bs[...]


def _lin4(r, wq, bq, wk, bk, wv, bv, ws, bs):
    sh = jax.ShapeDtypeStruct((NP, EMBD), _f32)
    return pl.pallas_call(
        _lin4_tc,
        compiler_params=_TC_PARAMS,
        out_shape=(sh, sh, sh, sh),
    )(r, wq, bq, wk, bk, wv, bv, ws, bs)


def _red_stats_tc(a_ref, q_ref, k_ref, o_ref):
    a = a_ref[...]
    s = _dot_nt(q_ref[...], k_ref[...]) * (1.0 / jnp.sqrt(float(EMBD)))
    s0 = jnp.sum(a * s)
    s1 = jnp.sum(a * s * s)
    s2 = jnp.sum(a)
    o_ref[...] = jnp.concatenate(
        [jnp.full((1, 1, 128), s0, _f32),
         jnp.full((1, 1, 128), s1, _f32),
         jnp.full((1, 1, 128), s2, _f32)], axis=2)


def _red_apply_tc(p_ref, a_ref, q_ref, k_ref, v_ref, sk_ref, o_ref):
    psum = jnp.sum(p_ref[...], axis=0)
    e_cnt = psum[0, 256]
    m1 = psum[0, 0] / e_cnt
    var = psum[0, 128] / e_cnt - m1 * m1
    scale = SCALE_PARAM * lax.rsqrt(var)
    a = a_ref[...]
    s = _dot_nt(q_ref[...], k_ref[...]) * (1.0 / jnp.sqrt(float(EMBD)))
    z = (s - m1) * scale
    w = 1.0 / (1.0 + jnp.exp(-z))
    o_ref[...] = _dot(a * w, v_ref[...]) + sk_ref[...]


def _reducer(A, q, k, v, sk):
    blk = pl.BlockSpec((BM, NP), lambda i: (i, 0))
    qblk = pl.BlockSpec((BM, EMBD), lambda i: (i, 0))
    kfull = pl.BlockSpec((NP, EMBD), lambda i: (0, 0))
    partials = pl.pallas_call(
        _red_stats_tc,
        compiler_params=_TC_PARAMS,
        grid=(GRID,),
        in_specs=[blk, qblk, kfull],
        out_specs=pl.BlockSpec((1, 1, 384), lambda i: (i, 0, 0)),
        out_shape=jax.ShapeDtypeStruct((GRID, 1, 384), _f32),
    )(A, q, k)
    return pl.pallas_call(
        _red_apply_tc,
        compiler_params=_TC_PARAMS,
        grid=(GRID,),
        in_specs=[pl.BlockSpec((GRID, 1, 384), lambda i: (0, 0, 0)),
                  blk, qblk, kfull, kfull, qblk],
        out_specs=qblk,
        out_shape=jax.ShapeDtypeStruct((NP, EMBD), _f32),
    )(partials, A, q, k, v, sk)


def _tconv_tc(a_ref, q_ref, k_ref, v_ref, sk_ref, o_ref):
    i = pl.program_id(0)
    a = a_ref[...]
    rows = lax.broadcasted_iota(jnp.int32, (BM, NP), 0) + i * BM
    cols = lax.broadcasted_iota(jnp.int32, (BM, NP), 1)
    ap = a + jnp.where(rows == cols, 1.0, 0.0)
    s = _dot_nt(q_ref[...], k_ref[...]) * (1.0 / jnp.sqrt(float(EMBD)))
    m = jnp.max(jnp.where(ap > 0, s, -1e30), axis=1, keepdims=True)
    p = ap * jnp.exp(s - m)
    den = jnp.sum(p, axis=1, keepdims=True)
    o_ref[...] = _dot(p / (den + 1e-16), v_ref[...]) + sk_ref[...]


def _tconv(A, q, k, v, sk):
    blk = pl.BlockSpec((BM, NP), lambda i: (i, 0))
    qblk = pl.BlockSpec((BM, EMBD), lambda i: (i, 0))
    kfull = pl.BlockSpec((NP, EMBD), lambda i: (0, 0))
    return pl.pallas_call(
        _tconv_tc,
        compiler_params=_TC_PARAMS,
        grid=(GRID,),
        in_specs=[blk, qblk, kfull, kfull, qblk],
        out_specs=qblk,
        out_shape=jax.ShapeDtypeStruct((NP, EMBD), _f32),
    )(A, q, k, v, sk)


def _tconv_ge_tc(a_ref, q_ref, k_ref, v_ref, sk_ref, mu_ref, nz_ref, o_ref):
    i = pl.program_id(0)
    a = a_ref[...]
    rows = lax.broadcasted_iota(jnp.int32, (BM, NP), 0) + i * BM
    cols = lax.broadcasted_iota(jnp.int32, (BM, NP), 1)
    ap = a + jnp.where(rows == cols, 1.0, 0.0)
    s = _dot_nt(q_ref[...], k_ref[...]) * (1.0 / jnp.sqrt(float(EMBD)))
    m = jnp.max(jnp.where(ap > 0, s, -1e30), axis=1, keepdims=True)
    p = ap * jnp.exp(s - m)
    den = jnp.sum(p, axis=1, keepdims=True)
    ls = _dot(p / (den + 1e-16), v_ref[...]) + sk_ref[...]
    ls = jnp.minimum(ls, MAX_LOGSTD)
    o_ref[...] = mu_ref[...] + nz_ref[...] * jnp.exp(ls)


def _tconv_ge(A, q, k, v, sk, mu, noise):
    blk = pl.BlockSpec((BM, NP), lambda i: (i, 0))
    qblk = pl.BlockSpec((BM, EMBD), lambda i: (i, 0))
    kfull = pl.BlockSpec((NP, EMBD), lambda i: (0, 0))
    return pl.pallas_call(
        _tconv_ge_tc,
        compiler_params=_TC_PARAMS,
        grid=(GRID,),
        in_specs=[blk, qblk, kfull, kfull, qblk, qblk, qblk],
        out_specs=qblk,
        out_shape=jax.ShapeDtypeStruct((NP, EMBD), _f32),
    )(A, q, k, v, sk, mu, noise)


def _dec1_tc(e_ref, w_ref, b_ref, g_ref, bb_ref, o_ref):
    out = _dot(e_ref[...], w_ref[...]) + b_ref[...]
    out = out * (1.0 / jnp.sqrt(1.0 + 1e-5)) * g_ref[...] + bb_ref[...]
    o_ref[...] = jnp.maximum(out, 0.0)


def _dec1(emb, w, b, g, bb):
    return pl.pallas_call(
        _dec1_tc,
        compiler_params=_TC_PARAMS,
        out_shape=jax.ShapeDtypeStruct((NP, C), _f32),
    )(emb, w, b, g, bb)


def _dec2_tc(x_ref, w_ref, b_ref, o_ref):
    o_ref[...] = jnp.maximum(_dot(x_ref[...], w_ref[...]) + b_ref[...], 0.0)


def _dec2(x, w, b):
    return pl.pallas_call(
        _dec2_tc,
        compiler_params=_TC_PARAMS,
        out_shape=jax.ShapeDtypeStruct((C, INTER), _f32),
    )(x, w, b)


def _final_tc(t_ref, w1_ref, ge_ref, w2_ref, b_ref, o_ref):
    acc = _dot(t_ref[...], w1_ref[...]) + _dot(ge_ref[...], w2_ref[...])
    o_ref[...] = jnp.maximum(acc + b_ref[...], 0.0)


def _final(t, w1, ge, w2, b):
    return pl.pallas_call(
        _final_tc,
        compiler_params=_TC_PARAMS,
        out_shape=jax.ShapeDtypeStruct((C, NP), _f32),
    )(t, w1, ge, w2, b)


def _pad_rr(a):
    return jnp.pad(a, ((0, NP - R), (0, NP - R)))


def _pad_r(a):
    return jnp.pad(a, ((0, NP - R),) + ((0, 0),) * (a.ndim - 1))


def kernel(x, knn_edge_index, ppi_edge_index,
           cols0_Wl, cols0_bl, cols0_Wr, cols1_Wl, cols1_bl, cols1_Wr,
           rows0_Wl, rows0_bl, rows0_Wr, rows1_Wl, rows1_bl, rows1_Wr,
           re_gcn_W, re_gcn_b, re_Wq, re_bq, re_Wk, re_bk, re_Wv, re_bv,
           re_Ws, re_bs, ce_gcn_W, ce_gcn_b,
           ce_mu_Wq, ce_mu_bq, ce_mu_Wk, ce_mu_bk, ce_mu_Wv, ce_mu_bv,
           ce_mu_Ws, ce_mu_bs,
           ce_ls_Wq, ce_ls_bq, ce_ls_Wk, ce_ls_bk, ce_ls_Wv, ce_ls_bv,
           ce_ls_Ws, ce_ls_bs,
           dec_W, dec_b, bn_gamma, bn_beta, transp_W, transp_b,
           comb_W, comb_b):
    row2 = lambda v: v.reshape(1, -1)
    A_knn, A_ppi = _build_adj(knn_edge_index, ppi_edge_index)

    Xp = _pad_r(x)
    H = _sage(A_knn, Xp.T, _pad_rr(cols0_Wl), row2(_pad_r(cols0_bl)),
              _pad_rr(cols0_Wr))
    H = _sage(A_ppi, H.T, rows0_Wl, row2(rows0_bl), rows0_Wr)
    H = _sage(A_knn, H.T, _pad_rr(cols1_Wl), row2(_pad_r(cols1_bl)),
              _pad_rr(cols1_Wr))
    H = _sage(A_ppi, H.T, rows1_Wl, row2(rows1_bl), rows1_Wr)

    r = _gcn(A_ppi, H, re_gcn_W, row2(re_gcn_b))
    c = _gcn(A_knn, H.T, _pad_r(ce_gcn_W), row2(ce_gcn_b))

    q, k, v, sk = _lin4(r, re_Wq, row2(re_bq), re_Wk, row2(re_bk),
                        re_Wv, row2(re_bv), re_Ws, row2(re_bs))
    row_emb = _reducer(A_ppi, q, k, v, sk)

    qm, km, vm, sm = _lin4(c, ce_mu_Wq, row2(ce_mu_bq), ce_mu_Wk,
                           row2(ce_mu_bk), ce_mu_Wv, row2(ce_mu_bv),
                           ce_mu_Ws, row2(ce_mu_bs))
    mu = _tconv(A_knn, qm, km, vm, sm)

    ql, kl, vl, sl = _lin4(c, ce_ls_Wq, row2(ce_ls_bq), ce_ls_Wk,
                           row2(ce_ls_bk), ce_ls_Wv, row2(ce_ls_bv),
                           ce_ls_Ws, row2(ce_ls_bs))
    noise = jax.random.normal(jax.random.key(42), (C, EMBD), _f32)
    ge = _tconv_ge(A_knn, ql, kl, vl, sl, mu, noise)

    out = _dec1(row_emb, dec_W, row2(dec_b), row2(bn_gamma),
                row2(bn_beta))
    t = _dec2(out.T, _pad_r(transp_W), row2(transp_b))
    w1 = jnp.pad(comb_W[:INTER], ((0, 0), (0, NP - R)))
    w2 = jnp.pad(comb_W[INTER:], ((0, 0), (0, NP - R)))
    cb = jnp.pad(comb_b, (0, NP - R))
    F = _final(t, w1, ge, w2, row2(cb))
    return F.T[:R]


# speedup vs baseline: 8.8702x; 1.9039x over previous
import functools

import jax
import jax.numpy as jnp
from jax import lax
from jax.experimental import pallas as pl
from jax.experimental.pallas import tpu as pltpu
from jax.experimental.pallas import tpu_sc as plsc

R = 2000
C = 2048
NP = 2048
E_PPI = 80000
E_KNN = 30720
INTER = 512
EMBD = 128
SCALE_PARAM = 3.0
MAX_LOGSTD = 10.0

ROWS = 32
N_TILES = 32
N_PASS = NP // (ROWS * N_TILES)
CH_PPI = 4000
CH_KNN = 3840

_f32 = jnp.float32


def _adj_body(ksrc_ref, kdst_ref, psrc_ref, pdst_ref, zer_ref,
              aknn_ref, appi_ref, esrc, edst, acc):
    cid = lax.axis_index("c")
    sid = lax.axis_index("s")
    wid = sid * 2 + cid
    ones = jnp.ones((16,), _f32)

    def scan_graph(src_ref, dst_ref, out_ref, n_edges, ch):
        nch = n_edges // ch
        for p in range(N_PASS):
            base = (p * N_TILES + wid) * ROWS
            pltpu.sync_copy(zer_ref, acc)

            def chunk_body(ci, _):
                off = ci * ch
                pltpu.sync_copy(src_ref.at[pl.ds(off, ch)],
                                esrc.at[pl.ds(0, ch)])
                pltpu.sync_copy(dst_ref.at[pl.ds(off, ch)],
                                edst.at[pl.ds(0, ch)])

                def step(j, carry):
                    s16 = esrc[pl.ds(j * 16, 16)]
                    d16 = edst[pl.ds(j * 16, 16)]
                    locr = d16 - base
                    msk = (locr >= 0) & (locr < ROWS)
                    flat = jnp.where(msk, locr * NP + s16, 0)
                    plsc.addupdate_scatter(acc, [flat], ones, mask=msk)
                    return carry

                lax.fori_loop(0, ch // 16, step, 0)
                return _

            lax.fori_loop(0, nch, chunk_body, 0)
            pltpu.sync_copy(acc, out_ref.at[pl.ds(base * NP, ROWS * NP)])

    scan_graph(ksrc_ref, kdst_ref, aknn_ref, E_KNN, CH_KNN)
    scan_graph(psrc_ref, pdst_ref, appi_ref, E_PPI, CH_PPI)


def _build_adj(knn_ei, ppi_ei):
    mesh = plsc.VectorSubcoreMesh(core_axis_name="c", subcore_axis_name="s")
    zer = jnp.zeros((ROWS * NP,), _f32)
    call = pl.kernel(
        _adj_body,
        out_type=(jax.ShapeDtypeStruct((NP * NP,), _f32),
                  jax.ShapeDtypeStruct((NP * NP,), _f32)),
        mesh=mesh,
        compiler_params=pltpu.CompilerParams(needs_layout_passes=False),
        scratch_types=[
            pltpu.VMEM((CH_PPI,), jnp.int32),
            pltpu.VMEM((CH_PPI,), jnp.int32),
            pltpu.VMEM((ROWS * NP,), _f32),
        ],
    )
    a_knn, a_ppi = call(knn_ei[0], knn_ei[1], ppi_ei[0], ppi_ei[1], zer)
    return a_knn.reshape(NP, NP), a_ppi.reshape(NP, NP)


def _leaky(x):
    return jnp.where(x >= 0, x, 0.01 * x)


def _dot(a, b, prec=lax.Precision.HIGHEST):
    return jnp.dot(a, b, preferred_element_type=_f32, precision=prec)


def _dot_fast(a, b):
    return jnp.dot(a, b, preferred_element_type=_f32,
                   precision=lax.Precision.DEFAULT)


def _dot_nt(a, b):
    return lax.dot_general(a, b, (((1,), (1,)), ((), ())),
                           preferred_element_type=_f32,
                           precision=lax.Precision.HIGHEST)


BM = 256
GRID = NP // BM
_TC_PARAMS = pltpu.CompilerParams(vmem_limit_bytes=100 * 1024 * 1024)


def _sage_mean_tc(a_ref, xf_ref, o_ref):
    a = a_ref[...]
    cnt = jnp.sum(a, axis=1, keepdims=True)
    o_ref[...] = _dot_fast(a, xf_ref[...]) / jnp.maximum(cnt, 1.0)


def _sage_comb_tc(m_ref, xb_ref, wl_ref, wr_ref, bl_ref, o_ref):
    acc = _dot_fast(m_ref[...], wl_ref[...]) + bl_ref[...]
    acc = acc + _dot_fast(xb_ref[...], wr_ref[...])
    o_ref[...] = _leaky(acc)


def _sage(A, X, Wl, bl, Wr):
    full = pl.BlockSpec((NP, NP), lambda i: (0, 0))
    blk = pl.BlockSpec((BM, NP), lambda i: (i, 0))
    mean = pl.pallas_call(
        _sage_mean_tc,
        grid=(GRID,),
        compiler_params=_TC_PARAMS,
        in_specs=[blk, full],
        out_specs=blk,
        out_shape=jax.ShapeDtypeStruct((NP, NP), _f32),
    )(A, X)
    return pl.pallas_call(
        _sage_comb_tc,
        grid=(GRID,),
        compiler_params=_TC_PARAMS,
        in_specs=[blk, blk, full, full,
                  pl.BlockSpec((1, NP), lambda i: (0, 0))],
        out_specs=blk,
        out_shape=jax.ShapeDtypeStruct((NP, NP), _f32),
    )(mean, X, Wl, Wr, bl)


def _gcn_tc(a_ref, h_ref, w_ref, b_ref, o_ref):
    a = a_ref[...]
    cnt = jnp.sum(a, axis=1, keepdims=True)
    dinv = lax.rsqrt(cnt + 1.0)
    g = _dot(h_ref[...], w_ref[...]) * dinv
    o_ref[...] = _leaky((_dot(a, g) + g) * dinv + b_ref[...])


def _gcn(A, H, W, b):
    return pl.pallas_call(
        _gcn_tc,
        compiler_params=_TC_PARAMS,
        out_shape=jax.ShapeDtypeStruct((NP, INTER), _f32),
    )(A, H, W, b)


def _lin4_tc(r_ref, wq, bq, wk, bk, wv, bv, ws, bs, oq, ok, ov, os):
    r = r_ref[...]
    oq[...] = _dot(r, wq[...]) + bq[...]
    ok[...] = _dot(r, wk[...]) + bk[...]
    ov[...] = _dot(r, wv[...]) + bv[...]
    os[...] = _dot(r, ws[...]) + bs[...]


def _lin4(r, wq, bq, wk, bk, wv, bv, ws, bs):
    sh = jax.ShapeDtypeStruct((NP, EMBD), _f32)
    return pl.pallas_call(
        _lin4_tc,
        compiler_params=_TC_PARAMS,
        out_shape=(sh, sh, sh, sh),
    )(r, wq, bq, wk, bk, wv, bv, ws, bs)


def _red_stats_tc(a_ref, q_ref, k_ref, o_ref):
    a = a_ref[...]
    s = _dot_nt(q_ref[...], k_ref[...]) * (1.0 / jnp.sqrt(float(EMBD)))
    s0 = jnp.sum(a * s)
    s1 = jnp.sum(a * s * s)
    s2 = jnp.sum(a)
    o_ref[...] = jnp.concatenate(
        [jnp.full((1, 1, 128), s0, _f32),
         jnp.full((1, 1, 128), s1, _f32),
         jnp.full((1, 1, 128), s2, _f32)], axis=2)


def _red_apply_tc(p_ref, a_ref, q_ref, k_ref, v_ref, sk_ref, o_ref):
    psum = jnp.sum(p_ref[...], axis=0)
    e_cnt = psum[0, 256]
    m1 = psum[0, 0] / e_cnt
    var = psum[0, 128] / e_cnt - m1 * m1
    scale = SCALE_PARAM * lax.rsqrt(var)
    a = a_ref[...]
    s = _dot_nt(q_ref[...], k_ref[...]) * (1.0 / jnp.sqrt(float(EMBD)))
    z = (s - m1) * scale
    w = 1.0 / (1.0 + jnp.exp(-z))
    o_ref[...] = _dot(a * w, v_ref[...]) + sk_ref[...]


def _reducer(A, q, k, v, sk):
    blk = pl.BlockSpec((BM, NP), lambda i: (i, 0))
    qblk = pl.BlockSpec((BM, EMBD), lambda i: (i, 0))
    kfull = pl.BlockSpec((NP, EMBD), lambda i: (0, 0))
    partials = pl.pallas_call(
        _red_stats_tc,
        compiler_params=_TC_PARAMS,
        grid=(GRID,),
        in_specs=[blk, qblk, kfull],
        out_specs=pl.BlockSpec((1, 1, 384), lambda i: (i, 0, 0)),
        out_shape=jax.ShapeDtypeStruct((GRID, 1, 384), _f32),
    )(A, q, k)
    return pl.pallas_call(
        _red_apply_tc,
        compiler_params=_TC_PARAMS,
        grid=(GRID,),
        in_specs=[pl.BlockSpec((GRID, 1, 384), lambda i: (0, 0, 0)),
                  blk, qblk, kfull, kfull, qblk],
        out_specs=qblk,
        out_shape=jax.ShapeDtypeStruct((NP, EMBD), _f32),
    )(partials, A, q, k, v, sk)


def _tconv_tc(a_ref, q_ref, k_ref, v_ref, sk_ref, o_ref):
    i = pl.program_id(0)
    a = a_ref[...]
    rows = lax.broadcasted_iota(jnp.int32, (BM, NP), 0) + i * BM
    cols = lax.broadcasted_iota(jnp.int32, (BM, NP), 1)
    ap = a + jnp.where(rows == cols, 1.0, 0.0)
    s = _dot_nt(q_ref[...], k_ref[...]) * (1.0 / jnp.sqrt(float(EMBD)))
    m = jnp.max(jnp.where(ap > 0, s, -1e30), axis=1, keepdims=True)
    p = ap * jnp.exp(s - m)
    den = jnp.sum(p, axis=1, keepdims=True)
    o_ref[...] = _dot(p / (den + 1e-16), v_ref[...]) + sk_ref[...]


def _tconv(A, q, k, v, sk):
    blk = pl.BlockSpec((BM, NP), lambda i: (i, 0))
    qblk = pl.BlockSpec((BM, EMBD), lambda i: (i, 0))
    kfull = pl.BlockSpec((NP, EMBD), lambda i: (0, 0))
    return pl.pallas_call(
        _tconv_tc,
        compiler_params=_TC_PARAMS,
        grid=(GRID,),
        in_specs=[blk, qblk, kfull, kfull, qblk],
        out_specs=qblk,
        out_shape=jax.ShapeDtypeStruct((NP, EMBD), _f32),
    )(A, q, k, v, sk)


def _tconv_ge_tc(a_ref, q_ref, k_ref, v_ref, sk_ref, mu_ref, nz_ref, o_ref):
    i = pl.program_id(0)
    a = a_ref[...]
    rows = lax.broadcasted_iota(jnp.int32, (BM, NP), 0) + i * BM
    cols = lax.broadcasted_iota(jnp.int32, (BM, NP), 1)
    ap = a + jnp.where(rows == cols, 1.0, 0.0)
    s = _dot_nt(q_ref[...], k_ref[...]) * (1.0 / jnp.sqrt(float(EMBD)))
    m = jnp.max(jnp.where(ap > 0, s, -1e30), axis=1, keepdims=True)
    p = ap * jnp.exp(s - m)
    den = jnp.sum(p, axis=1, keepdims=True)
    ls = _dot(p / (den + 1e-16), v_ref[...]) + sk_ref[...]
    ls = jnp.minimum(ls, MAX_LOGSTD)
    o_ref[...] = mu_ref[...] + nz_ref[...] * jnp.exp(ls)


def _tconv_ge(A, q, k, v, sk, mu, noise):
    blk = pl.BlockSpec((BM, NP), lambda i: (i, 0))
    qblk = pl.BlockSpec((BM, EMBD), lambda i: (i, 0))
    kfull = pl.BlockSpec((NP, EMBD), lambda i: (0, 0))
    return pl.pallas_call(
        _tconv_ge_tc,
        compiler_params=_TC_PARAMS,
        grid=(GRID,),
        in_specs=[blk, qblk, kfull, kfull, qblk, qblk, qblk],
        out_specs=qblk,
        out_shape=jax.ShapeDtypeStruct((NP, EMBD), _f32),
    )(A, q, k, v, sk, mu, noise)


def _dec1_tc(e_ref, w_ref, b_ref, g_ref, bb_ref, o_ref):
    out = _dot(e_ref[...], w_ref[...]) + b_ref[...]
    out = out * (1.0 / jnp.sqrt(1.0 + 1e-5)) * g_ref[...] + bb_ref[...]
    o_ref[...] = jnp.maximum(out, 0.0)


def _dec1(emb, w, b, g, bb):
    return pl.pallas_call(
        _dec1_tc,
        compiler_params=_TC_PARAMS,
        out_shape=jax.ShapeDtypeStruct((NP, C), _f32),
    )(emb, w, b, g, bb)


def _dec2_tc(x_ref, w_ref, b_ref, o_ref):
    o_ref[...] = jnp.maximum(_dot(x_ref[...], w_ref[...]) + b_ref[...], 0.0)


def _dec2(x, w, b):
    return pl.pallas_call(
        _dec2_tc,
        compiler_params=_TC_PARAMS,
        out_shape=jax.ShapeDtypeStruct((C, INTER), _f32),
    )(x, w, b)


def _final_tc(t_ref, w1_ref, ge_ref, w2_ref, b_ref, o_ref):
    acc = _dot(t_ref[...], w1_ref[...]) + _dot(ge_ref[...], w2_ref[...])
    o_ref[...] = jnp.maximum(acc + b_ref[...], 0.0)


def _final(t, w1, ge, w2, b):
    return pl.pallas_call(
        _final_tc,
        compiler_params=_TC_PARAMS,
        out_shape=jax.ShapeDtypeStruct((C, NP), _f32),
    )(t, w1, ge, w2, b)


def _pad_rr(a):
    return jnp.pad(a, ((0, NP - R), (0, NP - R)))


def _pad_r(a):
    return jnp.pad(a, ((0, NP - R),) + ((0, 0),) * (a.ndim - 1))


def kernel(x, knn_edge_index, ppi_edge_index,
           cols0_Wl, cols0_bl, cols0_Wr, cols1_Wl, cols1_bl, cols1_Wr,
           rows0_Wl, rows0_bl, rows0_Wr, rows1_Wl, rows1_bl, rows1_Wr,
           re_gcn_W, re_gcn_b, re_Wq, re_bq, re_Wk, re_bk, re_Wv, re_bv,
           re_Ws, re_bs, ce_gcn_W, ce_gcn_b,
           ce_mu_Wq, ce_mu_bq, ce_mu_Wk, ce_mu_bk, ce_mu_Wv, ce_mu_bv,
           ce_mu_Ws, ce_mu_bs,
           ce_ls_Wq, ce_ls_bq, ce_ls_Wk, ce_ls_bk, ce_ls_Wv, ce_ls_bv,
           ce_ls_Ws, ce_ls_bs,
           dec_W, dec_b, bn_gamma, bn_beta, transp_W, transp_b,
           comb_W, comb_b):
    row2 = lambda v: v.reshape(1, -1)
    A_knn, A_ppi = _build_adj(knn_edge_index, ppi_edge_index)

    Xp = _pad_r(x)
    H = _sage(A_knn, Xp.T, _pad_rr(cols0_Wl), row2(_pad_r(cols0_bl)),
              _pad_rr(cols0_Wr))
    H = _sage(A_ppi, H.T, rows0_Wl, row2(rows0_bl), rows0_Wr)
    H = _sage(A_knn, H.T, _pad_rr(cols1_Wl), row2(_pad_r(cols1_bl)),
              _pad_rr(cols1_Wr))
    H = _sage(A_ppi, H.T, rows1_Wl, row2(rows1_bl), rows1_Wr)

    r = _gcn(A_ppi, H, re_gcn_W, row2(re_gcn_b))
    c = _gcn(A_knn, H.T, _pad_r(ce_gcn_W), row2(ce_gcn_b))

    q, k, v, sk = _lin4(r, re_Wq, row2(re_bq), re_Wk, row2(re_bk),
                        re_Wv, row2(re_bv), re_Ws, row2(re_bs))
    row_emb = _reducer(A_ppi, q, k, v, sk)

    qm, km, vm, sm = _lin4(c, ce_mu_Wq, row2(ce_mu_bq), ce_mu_Wk,
                           row2(ce_mu_bk), ce_mu_Wv, row2(ce_mu_bv),
                           ce_mu_Ws, row2(ce_mu_bs))
    mu = _tconv(A_knn, qm, km, vm, sm)

    ql, kl, vl, sl = _lin4(c, ce_ls_Wq, row2(ce_ls_bq), ce_ls_Wk,
                           row2(ce_ls_bk), ce_ls_Wv, row2(ce_ls_bv),
                           ce_ls_Ws, row2(ce_ls_bs))
    noise = jax.random.normal(jax.random.key(42), (C, EMBD), _f32)
    ge = _tconv_ge(A_knn, ql, kl, vl, sl, mu, noise)

    out = _dec1(row_emb, dec_W, row2(dec_b), row2(bn_gamma),
                row2(bn_beta))
    t = _dec2(out.T, _pad_r(transp_W), row2(transp_b))
    w1 = jnp.pad(comb_W[:INTER], ((0, 0), (0, NP - R)))
    w2 = jnp.pad(comb_W[INTER:], ((0, 0), (0, NP - R)))
    cb = jnp.pad(comb_b, (0, NP - R))
    F = _final(t, w1, ge, w2, row2(cb))
    return F.T[:R]


# speedup vs baseline: 10.6991x; 1.2062x over previous
import functools

import jax
import jax.numpy as jnp
from jax import lax
from jax.experimental import pallas as pl
from jax.experimental.pallas import tpu as pltpu
from jax.experimental.pallas import tpu_sc as plsc

R = 2000
C = 2048
NP = 2048
E_PPI = 80000
E_KNN = 30720
INTER = 512
EMBD = 128
SCALE_PARAM = 3.0
MAX_LOGSTD = 10.0

ROWS = 32
N_TILES = 32
N_PASS = NP // (ROWS * N_TILES)
CH_PPI = 10000
CH_KNN = 7680

_f32 = jnp.float32


def _adj_body(ksrc_ref, kdst_ref, psrc_ref, pdst_ref, zer_ref,
              aknn_ref, appi_ref, esrc0, edst0, esrc1, edst1,
              acc, s0, s1, s2, s3):
    cid = lax.axis_index("c")
    sid = lax.axis_index("s")
    wid = sid * 2 + cid
    ones = jnp.ones((16,), _f32)
    sems = [(s0, s1), (s2, s3)]
    bufs = [(esrc0, edst0), (esrc1, edst1)]

    def scan_graph(src_ref, dst_ref, out_ref, n_edges, ch):
        nch = n_edges // ch

        def start(ci):
            b = ci % 2
            hs = pltpu.async_copy(src_ref.at[pl.ds(ci * ch, ch)],
                                  bufs[b][0].at[pl.ds(0, ch)], sems[b][0])
            hd = pltpu.async_copy(dst_ref.at[pl.ds(ci * ch, ch)],
                                  bufs[b][1].at[pl.ds(0, ch)], sems[b][1])
            return hs, hd

        for p in range(N_PASS):
            base = (p * N_TILES + wid) * ROWS
            pltpu.sync_copy(zer_ref, acc)
            pend = start(0)
            for ci in range(nch):
                b = ci % 2
                nxt = start(ci + 1) if ci + 1 < nch else None
                pend[0].wait()
                pend[1].wait()

                def step(j, carry):
                    s16 = bufs[b][0][pl.ds(j * 16, 16)]
                    d16 = bufs[b][1][pl.ds(j * 16, 16)]
                    locr = d16 - base
                    msk = (locr >= 0) & (locr < ROWS)
                    locr_c = jnp.where(msk, locr, 0)
                    plsc.addupdate_scatter(acc, [locr_c, s16], ones,
                                           mask=msk)
                    return carry

                lax.fori_loop(0, ch // 16, step, 0)
                pend = nxt
            pltpu.sync_copy(acc, out_ref.at[pl.ds(base, ROWS)])

    scan_graph(ksrc_ref, kdst_ref, aknn_ref, E_KNN, CH_KNN)
    scan_graph(psrc_ref, pdst_ref, appi_ref, E_PPI, CH_PPI)


def _build_adj(knn_ei, ppi_ei):
    mesh = plsc.VectorSubcoreMesh(core_axis_name="c", subcore_axis_name="s")
    zer = jnp.zeros((ROWS, NP), _f32)
    call = pl.kernel(
        _adj_body,
        out_type=(jax.ShapeDtypeStruct((NP, NP), _f32),
                  jax.ShapeDtypeStruct((NP, NP), _f32)),
        mesh=mesh,
        compiler_params=pltpu.CompilerParams(needs_layout_passes=False),
        scratch_types=[
            pltpu.VMEM((CH_PPI,), jnp.int32),
            pltpu.VMEM((CH_PPI,), jnp.int32),
            pltpu.VMEM((CH_PPI,), jnp.int32),
            pltpu.VMEM((CH_PPI,), jnp.int32),
            pltpu.VMEM((ROWS, NP), _f32),
            pltpu.SemaphoreType.DMA,
            pltpu.SemaphoreType.DMA,
            pltpu.SemaphoreType.DMA,
            pltpu.SemaphoreType.DMA,
        ],
    )
    return call(knn_ei[0], knn_ei[1], ppi_ei[0], ppi_ei[1], zer)


def _leaky(x):
    return jnp.where(x >= 0, x, 0.01 * x)


def _dot(a, b, prec=lax.Precision.HIGHEST):
    return jnp.dot(a, b, preferred_element_type=_f32, precision=prec)


def _dot_ref(a, b):
    return jnp.dot(a.astype(jnp.bfloat16), b.astype(jnp.bfloat16),
                   preferred_element_type=_f32)


def _dot_ax(a, x):
    ab = a.astype(jnp.bfloat16)
    xh = x.astype(jnp.bfloat16)
    xl = (x - xh.astype(_f32)).astype(jnp.bfloat16)
    return (jnp.dot(ab, xh, preferred_element_type=_f32)
            + jnp.dot(ab, xl, preferred_element_type=_f32))


def _dot_nt(a, b):
    return lax.dot_general(a, b, (((1,), (1,)), ((), ())),
                           preferred_element_type=_f32,
                           precision=lax.Precision.HIGHEST)


BM = 256
GRID = NP // BM
_TC_PARAMS = pltpu.CompilerParams(vmem_limit_bytes=100 * 1024 * 1024)


def _sage_mean_tc(a_ref, xf_ref, o_ref):
    a = a_ref[...]
    cnt = jnp.sum(a, axis=1, keepdims=True)
    o_ref[...] = _dot_ax(a, xf_ref[...]) / jnp.maximum(cnt, 1.0)


def _sage_comb_tc(m_ref, xb_ref, wl_ref, wr_ref, bl_ref, o_ref):
    acc = _dot_ref(m_ref[...], wl_ref[...]) + bl_ref[...]
    acc = acc + _dot_ref(xb_ref[...], wr_ref[...])
    o_ref[...] = _leaky(acc)


def _sage(A, X, Wl, bl, Wr):
    full = pl.BlockSpec((NP, NP), lambda i: (0, 0))
    blk = pl.BlockSpec((BM, NP), lambda i: (i, 0))
    mean = pl.pallas_call(
        _sage_mean_tc,
        grid=(GRID,),
        compiler_params=_TC_PARAMS,
        in_specs=[blk, full],
        out_specs=blk,
        out_shape=jax.ShapeDtypeStruct((NP, NP), _f32),
    )(A, X)
    return pl.pallas_call(
        _sage_comb_tc,
        grid=(GRID,),
        compiler_params=_TC_PARAMS,
        in_specs=[blk, blk, full, full,
                  pl.BlockSpec((1, NP), lambda i: (0, 0))],
        out_specs=blk,
        out_shape=jax.ShapeDtypeStruct((NP, NP), _f32),
    )(mean, X, Wl, Wr, bl)


def _gcn_tc(a_ref, h_ref, w_ref, b_ref, o_ref):
    a = a_ref[...]
    cnt = jnp.sum(a, axis=1, keepdims=True)
    dinv = lax.rsqrt(cnt + 1.0)
    g = _dot_ref(h_ref[...], w_ref[...]) * dinv
    o_ref[...] = _leaky((_dot_ax(a, g) + g) * dinv + b_ref[...])


def _gcn(A, H, W, b):
    return pl.pallas_call(
        _gcn_tc,
        compiler_params=_TC_PARAMS,
        out_shape=jax.ShapeDtypeStruct((NP, INTER), _f32),
    )(A, H, W, b)


def _lin4_tc(r_ref, wq, bq, wk, bk, wv, bv, ws, bs, oq, ok, ov, os):
    r = r_ref[...]
    oq[...] = _dot_ref(r, wq[...]) + bq[...]
    ok[...] = _dot_ref(r, wk[...]) + bk[...]
    ov[...] = _dot_ref(r, wv[...]) + bv[...]
    os[...] = _dot_ref(r, ws[...]) + bs[...]


def _lin4(r, wq, bq, wk, bk, wv, bv, ws, bs):
    sh = jax.ShapeDtypeStruct((NP, EMBD), _f32)
    return pl.pallas_call(
        _lin4_tc,
        compiler_params=_TC_PARAMS,
        out_shape=(sh, sh, sh, sh),
    )(r, wq, bq, wk, bk, wv, bv, ws, bs)


def _red_stats_tc(a_ref, q_ref, k_ref, o_ref):
    a = a_ref[...]
    s = _dot_nt(q_ref[...], k_ref[...]) * (1.0 / jnp.sqrt(float(EMBD)))
    s0 = jnp.sum(a * s)
    s1 = jnp.sum(a * s * s)
    s2 = jnp.sum(a)
    o_ref[...] = jnp.concatenate(
        [jnp.full((1, 1, 128), s0, _f32),
         jnp.full((1, 1, 128), s1, _f32),
         jnp.full((1, 1, 128), s2, _f32)], axis=2)


def _red_apply_tc(p_ref, a_ref, q_ref, k_ref, v_ref, sk_ref, o_ref):
    psum = jnp.sum(p_ref[...], axis=0)
    e_cnt = psum[0, 256]
    m1 = psum[0, 0] / e_cnt
    var = psum[0, 128] / e_cnt - m1 * m1
    scale = SCALE_PARAM * lax.rsqrt(var)
    a = a_ref[...]
    s = _dot_nt(q_ref[...], k_ref[...]) * (1.0 / jnp.sqrt(float(EMBD)))
    z = (s - m1) * scale
    w = 1.0 / (1.0 + jnp.exp(-z))
    o_ref[...] = _dot(a * w, v_ref[...]) + sk_ref[...]


def _reducer(A, q, k, v, sk):
    blk = pl.BlockSpec((BM, NP), lambda i: (i, 0))
    qblk = pl.BlockSpec((BM, EMBD), lambda i: (i, 0))
    kfull = pl.BlockSpec((NP, EMBD), lambda i: (0, 0))
    partials = pl.pallas_call(
        _red_stats_tc,
        compiler_params=_TC_PARAMS,
        grid=(GRID,),
        in_specs=[blk, qblk, kfull],
        out_specs=pl.BlockSpec((1, 1, 384), lambda i: (i, 0, 0)),
        out_shape=jax.ShapeDtypeStruct((GRID, 1, 384), _f32),
    )(A, q, k)
    return pl.pallas_call(
        _red_apply_tc,
        compiler_params=_TC_PARAMS,
        grid=(GRID,),
        in_specs=[pl.BlockSpec((GRID, 1, 384), lambda i: (0, 0, 0)),
                  blk, qblk, kfull, kfull, qblk],
        out_specs=qblk,
        out_shape=jax.ShapeDtypeStruct((NP, EMBD), _f32),
    )(partials, A, q, k, v, sk)


def _tconv_tc(a_ref, q_ref, k_ref, v_ref, sk_ref, o_ref):
    i = pl.program_id(0)
    a = a_ref[...]
    rows = lax.broadcasted_iota(jnp.int32, (BM, NP), 0) + i * BM
    cols = lax.broadcasted_iota(jnp.int32, (BM, NP), 1)
    ap = a + jnp.where(rows == cols, 1.0, 0.0)
    s = _dot_nt(q_ref[...], k_ref[...]) * (1.0 / jnp.sqrt(float(EMBD)))
    m = jnp.max(jnp.where(ap > 0, s, -1e30), axis=1, keepdims=True)
    p = ap * jnp.exp(s - m)
    den = jnp.sum(p, axis=1, keepdims=True)
    o_ref[...] = _dot(p / (den + 1e-16), v_ref[...]) + sk_ref[...]


def _tconv(A, q, k, v, sk):
    blk = pl.BlockSpec((BM, NP), lambda i: (i, 0))
    qblk = pl.BlockSpec((BM, EMBD), lambda i: (i, 0))
    kfull = pl.BlockSpec((NP, EMBD), lambda i: (0, 0))
    return pl.pallas_call(
        _tconv_tc,
        compiler_params=_TC_PARAMS,
        grid=(GRID,),
        in_specs=[blk, qblk, kfull, kfull, qblk],
        out_specs=qblk,
        out_shape=jax.ShapeDtypeStruct((NP, EMBD), _f32),
    )(A, q, k, v, sk)


def _tconv_ge_tc(a_ref, q_ref, k_ref, v_ref, sk_ref, mu_ref, nz_ref, o_ref):
    i = pl.program_id(0)
    a = a_ref[...]
    rows = lax.broadcasted_iota(jnp.int32, (BM, NP), 0) + i * BM
    cols = lax.broadcasted_iota(jnp.int32, (BM, NP), 1)
    ap = a + jnp.where(rows == cols, 1.0, 0.0)
    s = _dot_nt(q_ref[...], k_ref[...]) * (1.0 / jnp.sqrt(float(EMBD)))
    m = jnp.max(jnp.where(ap > 0, s, -1e30), axis=1, keepdims=True)
    p = ap * jnp.exp(s - m)
    den = jnp.sum(p, axis=1, keepdims=True)
    ls = _dot(p / (den + 1e-16), v_ref[...]) + sk_ref[...]
    ls = jnp.minimum(ls, MAX_LOGSTD)
    o_ref[...] = mu_ref[...] + nz_ref[...] * jnp.exp(ls)


def _tconv_ge(A, q, k, v, sk, mu, noise):
    blk = pl.BlockSpec((BM, NP), lambda i: (i, 0))
    qblk = pl.BlockSpec((BM, EMBD), lambda i: (i, 0))
    kfull = pl.BlockSpec((NP, EMBD), lambda i: (0, 0))
    return pl.pallas_call(
        _tconv_ge_tc,
        compiler_params=_TC_PARAMS,
        grid=(GRID,),
        in_specs=[blk, qblk, kfull, kfull, qblk, qblk, qblk],
        out_specs=qblk,
        out_shape=jax.ShapeDtypeStruct((NP, EMBD), _f32),
    )(A, q, k, v, sk, mu, noise)


def _dec1_tc(e_ref, w_ref, b_ref, g_ref, bb_ref, o_ref):
    out = _dot_ref(e_ref[...], w_ref[...]) + b_ref[...]
    out = out * (1.0 / jnp.sqrt(1.0 + 1e-5)) * g_ref[...] + bb_ref[...]
    o_ref[...] = jnp.maximum(out, 0.0)


def _dec1(emb, w, b, g, bb):
    return pl.pallas_call(
        _dec1_tc,
        compiler_params=_TC_PARAMS,
        out_shape=jax.ShapeDtypeStruct((NP, C), _f32),
    )(emb, w, b, g, bb)


def _dec2_tc(x_ref, w_ref, b_ref, o_ref):
    o_ref[...] = jnp.maximum(_dot_ref(x_ref[...], w_ref[...]) + b_ref[...], 0.0)


def _dec2(x, w, b):
    return pl.pallas_call(
        _dec2_tc,
        compiler_params=_TC_PARAMS,
        out_shape=jax.ShapeDtypeStruct((C, INTER), _f32),
    )(x, w, b)


def _final_tc(t_ref, w1_ref, ge_ref, w2_ref, b_ref, o_ref):
    acc = _dot_ref(t_ref[...], w1_ref[...]) + _dot_ref(ge_ref[...], w2_ref[...])
    o_ref[...] = jnp.maximum(acc + b_ref[...], 0.0)


def _final(t, w1, ge, w2, b):
    return pl.pallas_call(
        _final_tc,
        compiler_params=_TC_PARAMS,
        out_shape=jax.ShapeDtypeStruct((C, NP), _f32),
    )(t, w1, ge, w2, b)


def _pad_rr(a):
    return jnp.pad(a, ((0, NP - R), (0, NP - R)))


def _pad_r(a):
    return jnp.pad(a, ((0, NP - R),) + ((0, 0),) * (a.ndim - 1))


def kernel(x, knn_edge_index, ppi_edge_index,
           cols0_Wl, cols0_bl, cols0_Wr, cols1_Wl, cols1_bl, cols1_Wr,
           rows0_Wl, rows0_bl, rows0_Wr, rows1_Wl, rows1_bl, rows1_Wr,
           re_gcn_W, re_gcn_b, re_Wq, re_bq, re_Wk, re_bk, re_Wv, re_bv,
           re_Ws, re_bs, ce_gcn_W, ce_gcn_b,
           ce_mu_Wq, ce_mu_bq, ce_mu_Wk, ce_mu_bk, ce_mu_Wv, ce_mu_bv,
           ce_mu_Ws, ce_mu_bs,
           ce_ls_Wq, ce_ls_bq, ce_ls_Wk, ce_ls_bk, ce_ls_Wv, ce_ls_bv,
           ce_ls_Ws, ce_ls_bs,
           dec_W, dec_b, bn_gamma, bn_beta, transp_W, transp_b,
           comb_W, comb_b):
    row2 = lambda v: v.reshape(1, -1)
    A_knn, A_ppi = _build_adj(knn_edge_index, ppi_edge_index)

    Xp = _pad_r(x)
    H = _sage(A_knn, Xp.T, _pad_rr(cols0_Wl), row2(_pad_r(cols0_bl)),
              _pad_rr(cols0_Wr))
    H = _sage(A_ppi, H.T, rows0_Wl, row2(rows0_bl), rows0_Wr)
    H = _sage(A_knn, H.T, _pad_rr(cols1_Wl), row2(_pad_r(cols1_bl)),
              _pad_rr(cols1_Wr))
    H = _sage(A_ppi, H.T, rows1_Wl, row2(rows1_bl), rows1_Wr)

    r = _gcn(A_ppi, H, re_gcn_W, row2(re_gcn_b))
    c = _gcn(A_knn, H.T, _pad_r(ce_gcn_W), row2(ce_gcn_b))

    q, k, v, sk = _lin4(r, re_Wq, row2(re_bq), re_Wk, row2(re_bk),
                        re_Wv, row2(re_bv), re_Ws, row2(re_bs))
    row_emb = _reducer(A_ppi, q, k, v, sk)

    qm, km, vm, sm = _lin4(c, ce_mu_Wq, row2(ce_mu_bq), ce_mu_Wk,
                           row2(ce_mu_bk), ce_mu_Wv, row2(ce_mu_bv),
                           ce_mu_Ws, row2(ce_mu_bs))
    mu = _tconv(A_knn, qm, km, vm, sm)

    ql, kl, vl, sl = _lin4(c, ce_ls_Wq, row2(ce_ls_bq), ce_ls_Wk,
                           row2(ce_ls_bk), ce_ls_Wv, row2(ce_ls_bv),
                           ce_ls_Ws, row2(ce_ls_bs))
    noise = jax.random.normal(jax.random.key(42), (C, EMBD), _f32)
    ge = _tconv_ge(A_knn, ql, kl, vl, sl, mu, noise)

    out = _dec1(row_emb, dec_W, row2(dec_b), row2(bn_gamma),
                row2(bn_beta))
    t = _dec2(out.T, _pad_r(transp_W), row2(transp_b))
    w1 = jnp.pad(comb_W[:INTER], ((0, 0), (0, NP - R)))
    w2 = jnp.pad(comb_W[INTER:], ((0, 0), (0, NP - R)))
    cb = jnp.pad(comb_b, (0, NP - R))
    F = _final(t, w1, ge, w2, row2(cb))
    return F.T[:R]
